# baseline (speedup 1.0000x reference)
import functools
import jax
import jax.numpy as jnp
from jax import lax
from jax.experimental import pallas as pl
from jax.experimental.pallas import tpu as pltpu
from jax.experimental.pallas import tpu_sc as plsc

BATCH = 16384
DIM = 64

_info = plsc.get_sparse_core_info()
_NC, _NS = _info.num_cores, _info.num_subcores
_NW = _NC * _NS
_BPW = BATCH // _NW
_C = 64
_NCH = _BPW // _C


def _score_kernel(h, r, t, ent, rele, reln):
    mesh = plsc.VectorSubcoreMesh(core_axis_name="c", subcore_axis_name="s")

    @functools.partial(
        pl.kernel,
        mesh=mesh,
        out_type=jax.ShapeDtypeStruct((BATCH,), jnp.float32),
        compiler_params=pltpu.CompilerParams(
            needs_layout_passes=False, use_tc_tiling_on_sc=False),
        scratch_types=[
            pltpu.VMEM((_BPW,), jnp.int32),
            pltpu.VMEM((_BPW,), jnp.int32),
            pltpu.VMEM((_BPW,), jnp.int32),
            pltpu.VMEM((2, _C, DIM), jnp.float32),
            pltpu.VMEM((2, _C, DIM), jnp.float32),
            pltpu.VMEM((2, _C, DIM), jnp.float32),
            pltpu.VMEM((2, _C, DIM), jnp.float32),
            pltpu.VMEM((_BPW,), jnp.float32),
            pltpu.SemaphoreType.DMA,
            pltpu.SemaphoreType.DMA,
        ],
    )
    def k(h_hbm, r_hbm, t_hbm, ent_hbm, rele_hbm, reln_hbm, o_hbm,
          ihv, itv, irv, hbb, tbb, rbb, wbb, acc, s0, s1):
        wid = lax.axis_index("s") * _NC + lax.axis_index("c")
        base = wid * _BPW
        pltpu.sync_copy(h_hbm.at[pl.ds(base, _BPW)], ihv)
        pltpu.sync_copy(t_hbm.at[pl.ds(base, _BPW)], itv)
        pltpu.sync_copy(r_hbm.at[pl.ds(base, _BPW)], irv)

        sems = (s0, s1)

        def fire(c):
            p = c % 2
            sem = sems[p]
            coff = pl.ds(c * _C, _C)
            pltpu.async_copy(ent_hbm.at[ihv.at[coff]], hbb.at[p], sem)
            pltpu.async_copy(ent_hbm.at[itv.at[coff]], tbb.at[p], sem)
            pltpu.async_copy(rele_hbm.at[irv.at[coff]], rbb.at[p], sem)
            pltpu.async_copy(reln_hbm.at[irv.at[coff]], wbb.at[p], sem)

        def drain(c):
            p = c % 2
            coff = pl.ds(0, _C)
            pltpu.make_async_copy(ent_hbm.at[ihv.at[coff]],
                                  hbb.at[p], sems[p]).wait()
            pltpu.make_async_copy(ent_hbm.at[itv.at[coff]],
                                  tbb.at[p], sems[p]).wait()
            pltpu.make_async_copy(rele_hbm.at[irv.at[coff]],
                                  rbb.at[p], sems[p]).wait()
            pltpu.make_async_copy(reln_hbm.at[irv.at[coff]],
                                  wbb.at[p], sems[p]).wait()

        def compute(c):
            p = c % 2
            hb, tb, rb, wb = hbb.at[p], tbb.at[p], rbb.at[p], wbb.at[p]
            for g in range(_C // 16):
                sl = pl.ds(c * _C + g * 16, 16)
                elem = lax.iota(jnp.int32, 16) + g * 16
                zero = jnp.zeros((16,), jnp.float32)

                def dims(dd, carry):
                    sxx, sxw, sww = carry
                    d0 = dd * 8
                    for u in range(8):
                        d16 = jnp.full((16,), d0 + u, jnp.int32)
                        hd = plsc.load_gather(hb, [elem, d16])
                        td = plsc.load_gather(tb, [elem, d16])
                        rd = plsc.load_gather(rb, [elem, d16])
                        wd = plsc.load_gather(wb, [elem, d16])
                        x = hd + rd - td
                        sxx = sxx + x * x
                        sxw = sxw + x * wd
                        sww = sww + wd * wd
                    return (sxx, sxw, sww)

                sxx, sxw, sww = lax.fori_loop(
                    0, DIM // 8, dims, (zero, zero, zero))
                m2 = jnp.maximum(sww, 1e-24)
                val = jnp.maximum(sxx - (sxw * sxw) / m2, 0.0)
                bits = plsc.bitcast(val, jnp.int32)
                y = plsc.bitcast(
                    jnp.int32(0x5F3759DF) - lax.shift_right_logical(bits, 1),
                    jnp.float32)
                half = val * 0.5
                for _ in range(3):
                    y = y * (1.5 - half * y * y)
                acc[sl] = val * y

        fire(0)
        for c in range(_NCH):
            if c + 1 < _NCH:
                fire(c + 1)
            drain(c)
            compute(c)

        pltpu.sync_copy(acc, o_hbm.at[pl.ds(base, _BPW)])

    return k(h, r, t, ent, rele, reln)


@jax.jit
def kernel(h, r, t, ent_emb, rel_emb, rel_norm):
    return _score_kernel(h, r, t, ent_emb, rel_emb, rel_norm)


# speedup vs baseline: 2.1583x; 2.1583x over previous
import functools
import jax
import jax.numpy as jnp
from jax import lax
from jax.experimental import pallas as pl
from jax.experimental.pallas import tpu as pltpu
from jax.experimental.pallas import tpu_sc as plsc

BATCH = 16384
DIM = 64

_info = plsc.get_sparse_core_info()
_NC, _NS = _info.num_cores, _info.num_subcores
_NW = _NC * _NS
_BPW = BATCH // _NW
_C = 16
_NCH = _BPW // _C


def _score_kernel(h, r, t, ent3, rel2, reln2):
    mesh = plsc.VectorSubcoreMesh(core_axis_name="c", subcore_axis_name="s")

    @functools.partial(
        pl.kernel,
        mesh=mesh,
        out_type=jax.ShapeDtypeStruct((BATCH,), jnp.float32),
        compiler_params=pltpu.CompilerParams(needs_layout_passes=False),
        scratch_types=[
            pltpu.VMEM((_BPW + 16,), jnp.int32),
            pltpu.VMEM((_BPW + 16,), jnp.int32),
            pltpu.VMEM((_BPW,), jnp.int32),
            pltpu.VMEM((_BPW,), jnp.int32),
            pltpu.VMEM((_BPW + 16,), jnp.int32),
            pltpu.VMEM((_BPW + 16,), jnp.int32),
            pltpu.VMEM((2, _C, 8, DIM), jnp.float32),
            pltpu.VMEM((2, _C, 8, DIM), jnp.float32),
            pltpu.VMEM((2, _C, 2 * DIM), jnp.float32),
            pltpu.VMEM((2, _C, 2 * DIM), jnp.float32),
            pltpu.VMEM((_BPW,), jnp.float32),
            pltpu.SemaphoreType.DMA,
            pltpu.SemaphoreType.DMA,
            pltpu.SemaphoreType.DMA,
            pltpu.SemaphoreType.DMA,
        ],
    )
    def k(h_hbm, r_hbm, t_hbm, ent_hbm, rel_hbm, reln_hbm, o_hbm,
          ihv, itv, irv, ir2, iht, itt, hbb, tbb, rbb, wbb, acc, s0, s1, q0, q1):
        wid = lax.axis_index("s") * _NC + lax.axis_index("c")
        base = wid * _BPW
        pltpu.sync_copy(h_hbm.at[pl.ds(base, _BPW)], ihv.at[pl.ds(0, _BPW)])
        pltpu.sync_copy(t_hbm.at[pl.ds(base, _BPW)], itv.at[pl.ds(0, _BPW)])
        pltpu.sync_copy(r_hbm.at[pl.ds(base, _BPW)], irv)
        for j in range(_BPW // 16):
            sl = pl.ds(j * 16, 16)
            ir2[sl] = lax.shift_right_logical(irv[sl], 1)
            iht[sl] = lax.shift_right_logical(ihv[sl], 3)
            itt[sl] = lax.shift_right_logical(itv[sl], 3)

        sems = (s0, s1)
        rsems = (q0, q1)

        def fire(c):
            p = c % 2
            hb, tb = hbb.at[p], tbb.at[p]
            sem = sems[p]
            coff = c * _C

            def body(i, _):
                i4 = i * 4
                vh = iht[pl.ds(coff + i4, 16)]
                vt = itt[pl.ds(coff + i4, 16)]
                for u in range(4):
                    pltpu.async_copy(ent_hbm.at[pl.ds(vh[u], 1)],
                                     hb.at[pl.ds(i4 + u, 1)], sem)
                    pltpu.async_copy(ent_hbm.at[pl.ds(vt[u], 1)],
                                     tb.at[pl.ds(i4 + u, 1)], sem)
                return ()

            lax.fori_loop(0, _C // 4, body, ())
            pltpu.async_copy(rel_hbm.at[ir2.at[pl.ds(coff, _C)]],
                             rbb.at[p], rsems[p])
            pltpu.async_copy(reln_hbm.at[ir2.at[pl.ds(coff, _C)]],
                             wbb.at[p], rsems[p])

        def drain(c):
            p = c % 2
            pltpu.make_async_copy(ent_hbm.at[pl.ds(0, _C)],
                                  hbb.at[p], sems[p]).wait()
            pltpu.make_async_copy(ent_hbm.at[pl.ds(0, _C)],
                                  tbb.at[p], sems[p]).wait()
            pltpu.make_async_copy(rel_hbm.at[ir2.at[pl.ds(0, _C)]],
                                  rbb.at[p], rsems[p]).wait()
            pltpu.make_async_copy(reln_hbm.at[ir2.at[pl.ds(0, _C)]],
                                  wbb.at[p], rsems[p]).wait()

        def compute(c):
            p = c % 2
            hb, tb, rb, wb = hbb.at[p], tbb.at[p], rbb.at[p], wbb.at[p]
            coff = c * _C
            for g in range(_C // 16):
                sl = pl.ds(coff + g * 16, 16)
                hsub = lax.rem(ihv[sl], 8)
                tsub = lax.rem(itv[sl], 8)
                rpar = lax.mul(lax.rem(irv[sl], 2), DIM)
                elem = lax.iota(jnp.int32, 16) + g * 16
                zero = jnp.zeros((16,), jnp.float32)

                def dims(dd, carry):
                    sxx, sxw, sww = carry
                    d0 = dd * 8
                    for u in range(8):
                        d16 = jnp.full((16,), d0 + u, jnp.int32)
                        rw = d16 + rpar
                        hd = plsc.load_gather(hb, [elem, hsub, d16])
                        td = plsc.load_gather(tb, [elem, tsub, d16])
                        rd = plsc.load_gather(rb, [elem, rw])
                        wd = plsc.load_gather(wb, [elem, rw])
                        x = hd + rd - td
                        sxx = sxx + x * x
                        sxw = sxw + x * wd
                        sww = sww + wd * wd
                    return (sxx, sxw, sww)

                sxx, sxw, sww = lax.fori_loop(
                    0, DIM // 8, dims, (zero, zero, zero))
                m2 = jnp.maximum(sww, 1e-24)
                val = jnp.maximum(sxx - (sxw * sxw) / m2, 0.0)
                bits = plsc.bitcast(val, jnp.int32)
                y = plsc.bitcast(
                    jnp.int32(0x5F3759DF) - lax.shift_right_logical(bits, 1),
                    jnp.float32)
                half = val * 0.5
                for _ in range(3):
                    y = y * (1.5 - half * y * y)
                acc[sl] = val * y

        fire(0)
        for c in range(_NCH):
            if c + 1 < _NCH:
                fire(c + 1)
            drain(c)
            compute(c)

        pltpu.sync_copy(acc, o_hbm.at[pl.ds(base, _BPW)])

    return k(h, r, t, ent3, rel2, reln2)


@jax.jit
def kernel(h, r, t, ent_emb, rel_emb, rel_norm):
    ent3 = ent_emb.reshape(ent_emb.shape[0] // 8, 8, DIM)
    rel2 = rel_emb.reshape(rel_emb.shape[0] // 2, 2 * DIM)
    reln2 = rel_norm.reshape(rel_norm.shape[0] // 2, 2 * DIM)
    return _score_kernel(h, r, t, ent3, rel2, reln2)


# speedup vs baseline: 2.1770x; 1.0086x over previous
import functools
import jax
import jax.numpy as jnp
from jax import lax
from jax.experimental import pallas as pl
from jax.experimental.pallas import tpu as pltpu
from jax.experimental.pallas import tpu_sc as plsc

BATCH = 16384
DIM = 64

_info = plsc.get_sparse_core_info()
_NC, _NS = _info.num_cores, _info.num_subcores
_NW = _NC * _NS
_BPW = BATCH // _NW
_C = 16
_NCH = _BPW // _C


def _score_kernel(h, r, t, ent3, rel2, reln2):
    mesh = plsc.VectorSubcoreMesh(core_axis_name="c", subcore_axis_name="s")

    @functools.partial(
        pl.kernel,
        mesh=mesh,
        out_type=jax.ShapeDtypeStruct((BATCH,), jnp.float32),
        compiler_params=pltpu.CompilerParams(needs_layout_passes=False),
        scratch_types=[
            pltpu.VMEM((_BPW + 16,), jnp.int32),
            pltpu.VMEM((_BPW + 16,), jnp.int32),
            pltpu.VMEM((_BPW,), jnp.int32),
            pltpu.VMEM((_BPW,), jnp.int32),
            pltpu.VMEM((_BPW + 16,), jnp.int32),
            pltpu.VMEM((_BPW + 16,), jnp.int32),
            pltpu.VMEM((3, _C, 8, DIM), jnp.float32),
            pltpu.VMEM((3, _C, 8, DIM), jnp.float32),
            pltpu.VMEM((3, _C, 2 * DIM), jnp.float32),
            pltpu.VMEM((3, _C, 2 * DIM), jnp.float32),
            pltpu.VMEM((_BPW,), jnp.float32),
            pltpu.SemaphoreType.DMA,
            pltpu.SemaphoreType.DMA,
            pltpu.SemaphoreType.DMA,
            pltpu.SemaphoreType.DMA,
            pltpu.SemaphoreType.DMA,
            pltpu.SemaphoreType.DMA,
        ],
    )
    def k(h_hbm, r_hbm, t_hbm, ent_hbm, rel_hbm, reln_hbm, o_hbm,
          ihv, itv, irv, ir2, iht, itt, hbb, tbb, rbb, wbb, acc, s0, s1, s2, q0, q1, q2):
        wid = lax.axis_index("s") * _NC + lax.axis_index("c")
        base = wid * _BPW
        pltpu.sync_copy(h_hbm.at[pl.ds(base, _BPW)], ihv.at[pl.ds(0, _BPW)])
        pltpu.sync_copy(t_hbm.at[pl.ds(base, _BPW)], itv.at[pl.ds(0, _BPW)])
        pltpu.sync_copy(r_hbm.at[pl.ds(base, _BPW)], irv)
        for j in range(_BPW // 16):
            sl = pl.ds(j * 16, 16)
            ir2[sl] = lax.shift_right_logical(irv[sl], 1)
            iht[sl] = lax.shift_right_logical(ihv[sl], 3)
            itt[sl] = lax.shift_right_logical(itv[sl], 3)

        sems = (s0, s1, s2)
        rsems = (q0, q1, q2)

        def fire(c):
            p = c % 3
            hb, tb = hbb.at[p], tbb.at[p]
            sem = sems[p]
            coff = c * _C

            def body(i, _):
                i4 = i * 4
                vh = iht[pl.ds(coff + i4, 16)]
                vt = itt[pl.ds(coff + i4, 16)]
                for u in range(4):
                    pltpu.async_copy(ent_hbm.at[pl.ds(vh[u], 1)],
                                     hb.at[pl.ds(i4 + u, 1)], sem)
                    pltpu.async_copy(ent_hbm.at[pl.ds(vt[u], 1)],
                                     tb.at[pl.ds(i4 + u, 1)], sem)
                return ()

            lax.fori_loop(0, _C // 4, body, ())
            pltpu.async_copy(rel_hbm.at[ir2.at[pl.ds(coff, _C)]],
                             rbb.at[p], rsems[p])
            pltpu.async_copy(reln_hbm.at[ir2.at[pl.ds(coff, _C)]],
                             wbb.at[p], rsems[p])

        def drain(c):
            p = c % 3
            pltpu.make_async_copy(ent_hbm.at[pl.ds(0, _C)],
                                  hbb.at[p], sems[p]).wait()
            pltpu.make_async_copy(ent_hbm.at[pl.ds(0, _C)],
                                  tbb.at[p], sems[p]).wait()
            pltpu.make_async_copy(rel_hbm.at[ir2.at[pl.ds(0, _C)]],
                                  rbb.at[p], rsems[p]).wait()
            pltpu.make_async_copy(reln_hbm.at[ir2.at[pl.ds(0, _C)]],
                                  wbb.at[p], rsems[p]).wait()

        def compute(c):
            p = c % 3
            hb, tb, rb, wb = hbb.at[p], tbb.at[p], rbb.at[p], wbb.at[p]
            coff = c * _C
            for g in range(_C // 16):
                sl = pl.ds(coff + g * 16, 16)
                hsub = lax.rem(ihv[sl], 8)
                tsub = lax.rem(itv[sl], 8)
                rpar = lax.mul(lax.rem(irv[sl], 2), DIM)
                elem = lax.iota(jnp.int32, 16) + g * 16
                zero = jnp.zeros((16,), jnp.float32)

                def dims(dd, carry):
                    sxx, sxw, sww = carry
                    d0 = dd * 8
                    for u in range(8):
                        d16 = jnp.full((16,), d0 + u, jnp.int32)
                        rw = d16 + rpar
                        hd = plsc.load_gather(hb, [elem, hsub, d16])
                        td = plsc.load_gather(tb, [elem, tsub, d16])
                        rd = plsc.load_gather(rb, [elem, rw])
                        wd = plsc.load_gather(wb, [elem, rw])
                        x = hd + rd - td
                        sxx = sxx + x * x
                        sxw = sxw + x * wd
                        sww = sww + wd * wd
                    return (sxx, sxw, sww)

                sxx, sxw, sww = lax.fori_loop(
                    0, DIM // 8, dims, (zero, zero, zero))
                m2 = jnp.maximum(sww, 1e-24)
                val = jnp.maximum(sxx - (sxw * sxw) / m2, 0.0)
                bits = plsc.bitcast(val, jnp.int32)
                y = plsc.bitcast(
                    jnp.int32(0x5F3759DF) - lax.shift_right_logical(bits, 1),
                    jnp.float32)
                half = val * 0.5
                for _ in range(3):
                    y = y * (1.5 - half * y * y)
                acc[sl] = val * y

        fire(0)
        fire(1)
        for c in range(_NCH):
            if c + 2 < _NCH:
                fire(c + 2)
            drain(c)
            compute(c)

        pltpu.sync_copy(acc, o_hbm.at[pl.ds(base, _BPW)])

    return k(h, r, t, ent3, rel2, reln2)


@jax.jit
def kernel(h, r, t, ent_emb, rel_emb, rel_norm):
    ent3 = ent_emb.reshape(ent_emb.shape[0] // 8, 8, DIM)
    rel2 = rel_emb.reshape(rel_emb.shape[0] // 2, 2 * DIM)
    reln2 = rel_norm.reshape(rel_norm.shape[0] // 2, 2 * DIM)
    return _score_kernel(h, r, t, ent3, rel2, reln2)
